# baseline (speedup 1.0000x reference)
import functools

import jax
import jax.numpy as jnp
from jax import lax
from jax.experimental import pallas as pl
from jax.experimental.pallas import tpu as pltpu
from jax.experimental.pallas import tpu_sc as plsc

L = 16

_OW1 = 0
_OB1 = _OW1 + 128 * 32
_OW2 = _OB1 + 32
_OB2 = _OW2 + 32 * 32
_OW3 = _OB2 + 32
_OB3 = _OW3 + 32 * 64
_WTOT = _OB3 + 64


def _body(idx_hbm, table_hbm, w_hbm, out_hbm,
          idx_v, row0_v, row1_v, w_v, out_v, sem_w, sem_g0, sem_g1):
  cid = lax.axis_index("c")
  sid = lax.axis_index("s")

  @pl.when((cid == 0) & (sid == 0))
  def _():
    wcp = pltpu.async_copy(w_hbm, w_v, sem_w)
    pltpu.sync_copy(idx_hbm, idx_v)
    iv = idx_v[pl.ds(0, L)]
    i0 = iv[0]
    i1 = iv[1]
    g0 = pltpu.async_copy(table_hbm.at[pl.ds(i0, 1)], row0_v, sem_g0)
    g1 = pltpu.async_copy(table_hbm.at[pl.ds(i1, 1)], row1_v, sem_g1)
    g0.wait()
    g1.wait()
    wcp.wait()

    xv = ([row0_v[0, pl.ds(j * L, L)] for j in range(4)]
          + [row1_v[0, pl.ds(j * L, L)] for j in range(4)])

    acc1 = [w_v[pl.ds(_OB1 + c * L, L)] for c in range(2)]
    for k in range(128):
      s = xv[k // L][k % L]
      for c in range(2):
        acc1[c] = acc1[c] + s * w_v[pl.ds(_OW1 + k * 32 + c * L, L)]
    h1 = [jnp.maximum(a, 0.0) for a in acc1]

    acc2 = [w_v[pl.ds(_OB2 + c * L, L)] for c in range(2)]
    for k in range(32):
      s = h1[k // L][k % L]
      for c in range(2):
        acc2[c] = acc2[c] + s * w_v[pl.ds(_OW2 + k * 32 + c * L, L)]
    h2 = [jnp.maximum(a, 0.0) for a in acc2]

    acc3 = [w_v[pl.ds(_OB3 + c * L, L)] for c in range(4)]
    for k in range(32):
      s = h2[k // L][k % L]
      for c in range(4):
        acc3[c] = acc3[c] + s * w_v[pl.ds(_OW3 + k * 64 + c * L, L)]
    for c in range(4):
      out_v[0, pl.ds(c * L, L)] = jnp.maximum(acc3[c], 0.0)

    pltpu.sync_copy(out_v, out_hbm)


@jax.jit
def _run(idx16, table, w_flat):
  mesh = plsc.VectorSubcoreMesh(
      core_axis_name="c", subcore_axis_name="s", num_cores=2, num_subcores=16)
  return pl.kernel(
      _body,
      out_type=jax.ShapeDtypeStruct((1, 64), jnp.float32),
      mesh=mesh,
      scratch_types=[
          pltpu.VMEM((L,), jnp.int32),
          pltpu.VMEM((1, 64), jnp.float32),
          pltpu.VMEM((1, 64), jnp.float32),
          pltpu.VMEM((_WTOT,), jnp.float32),
          pltpu.VMEM((1, 64), jnp.float32),
          pltpu.SemaphoreType.DMA,
          pltpu.SemaphoreType.DMA,
          pltpu.SemaphoreType.DMA,
      ],
  )(idx16, table, w_flat)


def kernel(inputs, table, W1, b1, W2, b2, W3, b3):
  idx16 = jnp.zeros((L,), jnp.int32).at[:2].set(inputs.astype(jnp.int32))
  w_flat = jnp.concatenate([
      W1.T.reshape(-1), b1, W2.T.reshape(-1), b2, W3.T.reshape(-1), b3])
  return _run(idx16, table, w_flat)


# speedup vs baseline: 1.0509x; 1.0509x over previous
import functools

import jax
import jax.numpy as jnp
from jax import lax
from jax.experimental import pallas as pl
from jax.experimental.pallas import tpu as pltpu


def _tc_body(idx_ref, table_any, w1a, w1b, b1, w2, b2, w3, b3,
             out_ref, rows_v, sem0, sem1):
  i0 = idx_ref[0]
  i1 = idx_ref[1]
  cp0 = pltpu.make_async_copy(table_any.at[pl.ds(i0, 1)],
                              rows_v.at[pl.ds(0, 1)], sem0)
  cp0.start()
  cp1 = pltpu.make_async_copy(table_any.at[pl.ds(i1, 1)],
                              rows_v.at[pl.ds(1, 1)], sem1)
  cp1.start()
  cp0.wait()
  cp1.wait()
  r0 = rows_v[0:1, :]
  r1 = rows_v[1:2, :]
  h1 = jnp.maximum(
      jnp.dot(r0, w1a[...], preferred_element_type=jnp.float32)
      + jnp.dot(r1, w1b[...], preferred_element_type=jnp.float32)
      + b1[...], 0.0)
  h2 = jnp.maximum(
      jnp.dot(h1, w2[...], preferred_element_type=jnp.float32) + b2[...], 0.0)
  out_ref[...] = jnp.maximum(
      jnp.dot(h2, w3[...], preferred_element_type=jnp.float32) + b3[...], 0.0)


@jax.jit
def _run(idx, table, w1a, w1b, b1, w2, b2, w3, b3):
  grid_spec = pltpu.PrefetchScalarGridSpec(
      num_scalar_prefetch=1,
      grid=(1,),
      in_specs=[
          pl.BlockSpec(memory_space=pltpu.MemorySpace.HBM),
          pl.BlockSpec((64, 32), lambda i, idx: (0, 0)),
          pl.BlockSpec((64, 32), lambda i, idx: (0, 0)),
          pl.BlockSpec((1, 32), lambda i, idx: (0, 0)),
          pl.BlockSpec((32, 32), lambda i, idx: (0, 0)),
          pl.BlockSpec((1, 32), lambda i, idx: (0, 0)),
          pl.BlockSpec((32, 64), lambda i, idx: (0, 0)),
          pl.BlockSpec((1, 64), lambda i, idx: (0, 0)),
      ],
      out_specs=pl.BlockSpec((1, 64), lambda i, idx: (0, 0)),
      scratch_shapes=[
          pltpu.VMEM((2, 64), jnp.float32),
          pltpu.SemaphoreType.DMA,
          pltpu.SemaphoreType.DMA,
      ],
  )
  return pl.pallas_call(
      _tc_body,
      grid_spec=grid_spec,
      out_shape=jax.ShapeDtypeStruct((1, 64), jnp.float32),
  )(idx, table, w1a, w1b, b1, w2, b2, w3, b3)


def kernel(inputs, table, W1, b1, W2, b2, W3, b3):
  idx = inputs.astype(jnp.int32)
  w1t = W1.T
  return _run(idx, table, w1t[:64], w1t[64:], b1.reshape(1, 32),
              W2.T, b2.reshape(1, 32), W3.T, b3.reshape(1, 64))


# speedup vs baseline: 15.0140x; 14.2861x over previous
import functools

import jax
import jax.numpy as jnp
from jax import lax
from jax.experimental import pallas as pl
from jax.experimental.pallas import tpu as pltpu
from jax.experimental.pallas import tpu_sc as plsc

L = 16

_OW1 = 0
_OB1 = _OW1 + 128 * 32
_OW2 = _OB1 + 32
_OB2 = _OW2 + 32 * 32
_OW3 = _OB2 + 32
_OB3 = _OW3 + 32 * 64
_WTOT = _OB3 + 64


def _body(idx_hbm, tableT_hbm, w_hbm, out_hbm,
          idx_v, blk0_v, blk1_v, w_v, out_v, sem_w, sem_g0, sem_g1):
  cid = lax.axis_index("c")
  sid = lax.axis_index("s")

  @pl.when((cid == 0) & (sid == 0))
  def _():
    wcp = pltpu.async_copy(w_hbm, w_v, sem_w)
    pltpu.sync_copy(idx_hbm, idx_v)
    iv = idx_v[pl.ds(0, L)]
    i0 = iv[0]
    i1 = iv[1]
    a0 = pl.multiple_of((i0 // 128) * 128, 128)
    a1 = pl.multiple_of((i1 // 128) * 128, 128)
    g0 = pltpu.async_copy(tableT_hbm.at[:, pl.ds(a0, 128)], blk0_v, sem_g0)
    g1 = pltpu.async_copy(tableT_hbm.at[:, pl.ds(a1, 128)], blk1_v, sem_g1)
    g0.wait()
    g1.wait()
    wcp.wait()

    lanes = lax.broadcasted_iota(jnp.int32, (L,), 0)
    r0 = jnp.full((L,), i0 - a0, dtype=jnp.int32)
    r1 = jnp.full((L,), i1 - a1, dtype=jnp.int32)
    xv = ([plsc.load_gather(blk0_v, [c * L + lanes, r0]) for c in range(4)]
          + [plsc.load_gather(blk1_v, [c * L + lanes, r1]) for c in range(4)])

    acc1 = [w_v[pl.ds(_OB1 + c * L, L)] for c in range(2)]
    for k in range(128):
      s = xv[k // L][k % L]
      for c in range(2):
        acc1[c] = acc1[c] + s * w_v[pl.ds(_OW1 + k * 32 + c * L, L)]
    h1 = [jnp.maximum(a, 0.0) for a in acc1]

    acc2 = [w_v[pl.ds(_OB2 + c * L, L)] for c in range(2)]
    for k in range(32):
      s = h1[k // L][k % L]
      for c in range(2):
        acc2[c] = acc2[c] + s * w_v[pl.ds(_OW2 + k * 32 + c * L, L)]
    h2 = [jnp.maximum(a, 0.0) for a in acc2]

    acc3 = [w_v[pl.ds(_OB3 + c * L, L)] for c in range(4)]
    for k in range(32):
      s = h2[k // L][k % L]
      for c in range(4):
        acc3[c] = acc3[c] + s * w_v[pl.ds(_OW3 + k * 64 + c * L, L)]
    for c in range(4):
      out_v[0, pl.ds(c * L, L)] = jnp.maximum(acc3[c], 0.0)

    pltpu.sync_copy(out_v, out_hbm)


@jax.jit
def _run(idx16, tableT, w_flat):
  mesh = plsc.VectorSubcoreMesh(
      core_axis_name="c", subcore_axis_name="s", num_cores=2, num_subcores=16)
  return pl.kernel(
      _body,
      out_type=jax.ShapeDtypeStruct((1, 64), jnp.float32),
      mesh=mesh,
      scratch_types=[
          pltpu.VMEM((L,), jnp.int32),
          pltpu.VMEM((64, 128), jnp.float32),
          pltpu.VMEM((64, 128), jnp.float32),
          pltpu.VMEM((_WTOT,), jnp.float32),
          pltpu.VMEM((1, 64), jnp.float32),
          pltpu.SemaphoreType.DMA,
          pltpu.SemaphoreType.DMA,
          pltpu.SemaphoreType.DMA,
      ],
      compiler_params=pltpu.CompilerParams(needs_layout_passes=False),
  )(idx16, tableT, w_flat)


def kernel(inputs, table, W1, b1, W2, b2, W3, b3):
  idx16 = jnp.zeros((L,), jnp.int32).at[:2].set(inputs.astype(jnp.int32))
  w_flat = jnp.concatenate([
      W1.T.reshape(-1), b1, W2.T.reshape(-1), b2, W3.T.reshape(-1), b3])
  return _run(idx16, table.T, w_flat)
